# Initial kernel scaffold; baseline (speedup 1.0000x reference)
import jax
import jax.numpy as jnp
from jax.experimental import pallas as pl


def kernel(x, table):
    raise NotImplementedError("write your pallas kernel here")


# speedup vs baseline: 3.3311x; 3.3311x over previous
import functools

import jax
import jax.numpy as jnp
from jax import lax
from jax.experimental import pallas as pl
from jax.experimental.pallas import tpu as pltpu
from jax.experimental.pallas import tpu_sc as plsc

B = 16384 * 50
EMB = 32
NC, NS = 2, 16
NW = NC * NS
PER_W = B // NW
CHUNK = 128
NCHUNK = PER_W // CHUNK


@functools.partial(
    pl.kernel,
    mesh=plsc.VectorSubcoreMesh(core_axis_name="c", subcore_axis_name="s"),
    out_type=jax.ShapeDtypeStruct((NW, NCHUNK, CHUNK, EMB), jnp.float32),
    scratch_types=[
        pltpu.VMEM((NCHUNK, CHUNK), jnp.int32),
        pltpu.VMEM((CHUNK, EMB), jnp.float32),
        pltpu.SemaphoreType.DMA,
    ],
    compiler_params=pltpu.CompilerParams(use_tc_tiling_on_sc=False),
)
def _gather(x_hbm, table_hbm, out_hbm, idx_v, rows_v, sem):
    wid = lax.axis_index("s") * NC + lax.axis_index("c")
    pltpu.sync_copy(x_hbm.at[wid], idx_v)

    def body(c, _):
        pltpu.async_copy(table_hbm.at[idx_v.at[c]], rows_v, sem).wait()
        pltpu.sync_copy(rows_v, out_hbm.at[wid, c])
        return ()

    lax.fori_loop(0, NCHUNK, body, ())


def kernel(x, table):
    xr = x.reshape(NW, NCHUNK, CHUNK)
    out = _gather(xr, table)
    return out.reshape(1, -1)


# speedup vs baseline: 4.1467x; 1.2448x over previous
import functools

import jax
import jax.numpy as jnp
from jax import lax
from jax.experimental import pallas as pl
from jax.experimental.pallas import tpu as pltpu
from jax.experimental.pallas import tpu_sc as plsc

B = 16384 * 50
EMB = 32
NC, NS = 2, 16
NW = NC * NS
PER_W = B // NW
CHUNK = 128
NCHUNK = PER_W // CHUNK
KB = 10
NBATCH = NCHUNK // KB
NBUF = 2


@functools.partial(
    pl.kernel,
    mesh=plsc.VectorSubcoreMesh(core_axis_name="c", subcore_axis_name="s"),
    out_type=jax.ShapeDtypeStruct((NW, NBATCH, KB, CHUNK, EMB), jnp.float32),
    scratch_types=[
        pltpu.VMEM((NCHUNK, CHUNK), jnp.int32),
        pltpu.VMEM((NBUF, KB, CHUNK, EMB), jnp.float32),
        pltpu.SemaphoreType.DMA,
        pltpu.SemaphoreType.DMA,
        pltpu.SemaphoreType.DMA,
    ],
    compiler_params=pltpu.CompilerParams(use_tc_tiling_on_sc=False),
)
def _gather(x_hbm, table_hbm, out_hbm, idx_v, rows_v, gsem0, gsem1, ssem):
    wid = lax.axis_index("s") * NC + lax.axis_index("c")
    gsems = (gsem0, gsem1)
    pltpu.sync_copy(x_hbm.at[wid], idx_v)

    def fire_batch(b, buf):
        for j in range(KB):
            pltpu.async_copy(
                table_hbm.at[idx_v.at[b * KB + j]], rows_v.at[buf, j],
                gsems[buf])

    def drain_batch(buf):
        pltpu.make_async_copy(
            out_hbm.at[wid, 0], rows_v.at[buf], gsems[buf]).wait()

    fire_batch(0, 0)

    @pl.loop(0, NBATCH, step=NBUF)
    def _(g):
        for buf in range(NBUF):
            b = g + buf
            drain_batch(buf)

            @pl.when(b >= 1)
            def _():
                pltpu.make_async_copy(
                    rows_v.at[buf ^ 1], out_hbm.at[wid, 0], ssem).wait()

            @pl.when(b + 1 < NBATCH)
            def _():
                fire_batch(b + 1, buf ^ 1)

            pltpu.async_copy(rows_v.at[buf], out_hbm.at[wid, b], ssem)

    pltpu.make_async_copy(
        rows_v.at[(NBATCH - 1) % NBUF], out_hbm.at[wid, 0], ssem).wait()


def kernel(x, table):
    xr = x.reshape(NW, NCHUNK, CHUNK)
    out = _gather(xr, table)
    return out.reshape(1, -1)


# speedup vs baseline: 4.1559x; 1.0022x over previous
import functools

import jax
import jax.numpy as jnp
from jax import lax
from jax.experimental import pallas as pl
from jax.experimental.pallas import tpu as pltpu
from jax.experimental.pallas import tpu_sc as plsc

B = 16384 * 50
EMB = 32
NC, NS = 2, 16
NW = NC * NS
PER_W = B // NW
CHUNK = 128
NCHUNK = PER_W // CHUNK
KB = 10
NBATCH = NCHUNK // KB
NBUF = 2
BROWS = KB * CHUNK


@functools.partial(
    pl.kernel,
    mesh=plsc.VectorSubcoreMesh(core_axis_name="c", subcore_axis_name="s"),
    out_type=jax.ShapeDtypeStruct((B, EMB), jnp.float32),
    scratch_types=[
        pltpu.VMEM((NCHUNK, CHUNK), jnp.int32),
        pltpu.VMEM((NBUF, BROWS, EMB), jnp.float32),
        pltpu.SemaphoreType.DMA,
        pltpu.SemaphoreType.DMA,
        pltpu.SemaphoreType.DMA,
    ],
    compiler_params=pltpu.CompilerParams(use_tc_tiling_on_sc=False),
)
def _gather(x_hbm, table_hbm, out_hbm, idx_v, rows_v, gsem0, gsem1, ssem):
    wid = lax.axis_index("s") * NC + lax.axis_index("c")
    gsems = (gsem0, gsem1)
    pltpu.sync_copy(x_hbm.at[wid], idx_v)

    def out_slab(b):
        return out_hbm.at[pl.ds((wid * NBATCH + b) * BROWS, BROWS)]

    def fire_batch(b, buf):
        for j in range(KB):
            pltpu.async_copy(
                table_hbm.at[idx_v.at[b * KB + j]],
                rows_v.at[buf, pl.ds(j * CHUNK, CHUNK)],
                gsems[buf])

    def drain_batch(buf):
        pltpu.make_async_copy(out_slab(0), rows_v.at[buf], gsems[buf]).wait()

    fire_batch(0, 0)

    @pl.loop(0, NBATCH, step=NBUF)
    def _(g):
        for buf in range(NBUF):
            b = g + buf
            drain_batch(buf)

            @pl.when(b >= 1)
            def _():
                pltpu.make_async_copy(
                    rows_v.at[buf ^ 1], out_slab(0), ssem).wait()

            @pl.when(b + 1 < NBATCH)
            def _():
                fire_batch(b + 1, buf ^ 1)

            pltpu.async_copy(rows_v.at[buf], out_slab(b), ssem)

    pltpu.make_async_copy(
        rows_v.at[(NBATCH - 1) % NBUF], out_slab(0), ssem).wait()


def kernel(x, table):
    xr = x.reshape(NW, NCHUNK, CHUNK)
    out = _gather(xr, table)
    return out.reshape(1, -1)
